# baseline (speedup 1.0000x reference)
import functools
import numpy as np
import jax
import jax.numpy as jnp
from jax import lax
from jax.experimental import pallas as pl
from jax.experimental.pallas import tpu as pltpu, tpu_sc as plsc

_N = 10000
_E = 320000
_G = 128
_H1 = 128
_H2 = 64
_R = 8
_HEADS = 4
_NC = 2
_NS = 16

_D4 = 40192


def _splat(vec16, j):
    return jnp.broadcast_to(vec16[j], (16,))


def _fill(ref, nvec, value):
    v = jnp.full((16,), value, jnp.float32)
    if len(ref.shape) == 1:
        def _b(i, _):
            ref[pl.ds(i * 16, 16)] = v
            return 0
    else:
        ncol = ref.shape[1] // 16

        def _b(i, _):
            ref[i // ncol, pl.ds((i % ncol) * 16, 16)] = v
            return 0
    lax.fori_loop(0, nvec, _b, 0)


def _zero_rows(zb8, sp_ref, s):
    def _b(j, _):
        pltpu.sync_copy(zb8, sp_ref.at[pl.ds(s * 624 + j * 8, 8)])
        return 0
    lax.fori_loop(0, 78, _b, 0)

    @pl.when(s == _NS - 1)
    def _():
        pltpu.sync_copy(zb8, sp_ref.at[pl.ds(9984, 8)])
        pltpu.sync_copy(zb8, sp_ref.at[pl.ds(9992, 8)])


def _tile_rows(sync_fn, s):
    def _b(j, _):
        sync_fn(s * 624 + j * 208, 208)
        return 0
    lax.fori_loop(0, 3, _b, 0)

    @pl.when(s == _NS - 1)
    def _():
        sync_fn(9984, 16)


def _tc_matmul_body(x_ref, w_ref, o_ref):
    o_ref[...] = jnp.dot(x_ref[...], w_ref[...],
                         preferred_element_type=jnp.float32)


def _tc_matmul(x, w, bn=2000):
    n, kdim = x.shape
    m = w.shape[1]
    return pl.pallas_call(
        _tc_matmul_body,
        grid=(n // bn,),
        in_specs=[pl.BlockSpec((bn, kdim), lambda i: (i, 0)),
                  pl.BlockSpec((kdim, m), lambda i: (0, 0))],
        out_specs=pl.BlockSpec((bn, m), lambda i: (i, 0)),
        out_shape=jax.ShapeDtypeStruct((n, m), jnp.float32),
    )(x, w)


def _tc_fuse_body(a0_ref, a1_ref, xr_ref, b_ref, w_ref, b2_ref, o_ref):
    x2 = a0_ref[...] + a1_ref[...] + xr_ref[...] + b_ref[...]
    o_ref[...] = jnp.dot(x2, w_ref[...],
                         preferred_element_type=jnp.float32) + b2_ref[...]


def _tc_fuse(a0, a1, xr, b, w, b2, bn=2000):
    n, kdim = a0.shape
    m = w.shape[1]
    return pl.pallas_call(
        _tc_fuse_body,
        grid=(n // bn,),
        in_specs=[pl.BlockSpec((bn, kdim), lambda i: (i, 0)),
                  pl.BlockSpec((bn, kdim), lambda i: (i, 0)),
                  pl.BlockSpec((bn, kdim), lambda i: (i, 0)),
                  pl.BlockSpec((1, kdim), lambda i: (0, 0)),
                  pl.BlockSpec((kdim, m), lambda i: (0, 0)),
                  pl.BlockSpec((1, m), lambda i: (0, 0))],
        out_specs=pl.BlockSpec((bn, m), lambda i: (i, 0)),
        out_shape=jax.ShapeDtypeStruct((n, m), jnp.float32),
    )(a0, a1, xr, b, w, b2)


def _sc_rgcn_body(srcv, dstv, etv, tbl, aggp,
                  cnt_sp, agg_sp, onesb, zb2, zbd,
                  dstb, etb, idxc,
                  srcbA, etb2A, dstb2A, idxmA, idxc2A, cntgA, msgA,
                  srcbB, etb2B, dstb2B, idxmB, idxc2B, cntgB, msgB,
                  semA, semB):
    c = lax.axis_index("c")
    s = lax.axis_index("s")

    _fill(onesb, 125, 1.0)
    _fill(zb2, 64, 0.0)
    _fill(zbd, 157, 0.0)

    pltpu.sync_copy(zbd, cnt_sp.at[pl.ds(s * 5008, 2512)])
    pltpu.sync_copy(zbd, cnt_sp.at[pl.ds(s * 5008 + 2496, 2512)])
    _zero_rows(zb2, agg_sp, s)
    plsc.subcore_barrier()

    def _cnt(j, _):
        b0 = s * 20000 + j * 2000
        pltpu.sync_copy(dstv.at[pl.ds(b0, 2000)], dstb)
        pltpu.sync_copy(etv.at[pl.ds(b0, 2000)], etb)

        def _ix(i, _):
            sl = pl.ds(i * 16, 16)
            idxc[sl] = dstb[sl] * _R + etb[sl]
            return 0
        lax.fori_loop(0, 125, _ix, 0)
        pltpu.sync_copy(onesb, cnt_sp.at[idxc], add=True)
        return 0
    lax.fori_loop(0, 10, _cnt, 0)
    plsc.subcore_barrier()

    bufs = ((srcbA, etb2A, dstb2A, idxmA, idxc2A, cntgA, msgA, semA),
            (srcbB, etb2B, dstb2B, idxmB, idxc2B, cntgB, msgB, semB))

    def _start(cix, bi):
        srcb, etb2, dstb2, idxm, idxc2, cntg, msg, sem = bufs[bi]
        b0 = c * 160000 + s * 10000 + cix * 80
        pltpu.sync_copy(srcv.at[pl.ds(b0, 80)], srcb)
        pltpu.sync_copy(etv.at[pl.ds(b0, 80)], etb2)
        pltpu.sync_copy(dstv.at[pl.ds(b0, 80)], dstb2)

        def _ix2(i, _):
            sl = pl.ds(i * 16, 16)
            idxm[sl] = srcb[sl] * _R + etb2[sl]
            idxc2[sl] = dstb2[sl] * _R + etb2[sl]
            return 0
        lax.fori_loop(0, 5, _ix2, 0)
        pltpu.async_copy(tbl.at[idxm], msg, sem)

    def _wait(bi):
        srcb, etb2, dstb2, idxm, idxc2, cntg, msg, sem = bufs[bi]
        pltpu.make_async_copy(tbl.at[idxm], msg, sem).wait()
        pltpu.async_copy(cnt_sp.at[idxc2], cntg, sem).wait()

    def _compute(bi):
        srcb, etb2, dstb2, idxm, idxc2, cntg, msg, sem = bufs[bi]

        def _scale(i, _):
            c16 = cntg[pl.ds(i * 16, 16)]
            n16 = 1.0 / jnp.maximum(c16, 1.0)
            for jj in range(16):
                e = i * 16 + jj
                nv = _splat(n16, jj)
                for vv in range(8):
                    sl = pl.ds(vv * 16, 16)
                    msg[e, sl] = msg[e, sl] * nv
            return 0
        lax.fori_loop(0, 5, _scale, 0)
        pltpu.sync_copy(msg, agg_sp.at[dstb2], add=True)

    _start(0, 0)

    def _pair(i, _):
        _wait(0)
        _start(2 * i + 1, 1)
        _compute(0)
        _wait(1)
        _start(2 * i + 2, 0)
        _compute(1)
        return 0
    lax.fori_loop(0, 62, _pair, 0)
    _wait(0)
    _compute(0)
    plsc.subcore_barrier()

    _tile_rows(lambda r0, nr: pltpu.sync_copy(
        agg_sp.at[pl.ds(r0, nr)], aggp.at[c, pl.ds(r0, nr)]), s)


def _sc_rgcn(srcv, dstv, etv, tbl):
    mesh = plsc.VectorSubcoreMesh(core_axis_name="c", subcore_axis_name="s")
    kfn = functools.partial(
        pl.kernel,
        out_type=jax.ShapeDtypeStruct((_NC, _N, _H1), jnp.float32),
        mesh=mesh,
        scratch_types=[
            pltpu.VMEM_SHARED((80128,), jnp.float32),
            pltpu.VMEM_SHARED((_N, _H1), jnp.float32),
            pltpu.VMEM((2000,), jnp.float32),
            pltpu.VMEM((8, _H1), jnp.float32),
            pltpu.VMEM((2512,), jnp.float32),
            pltpu.VMEM((2000,), jnp.int32),
            pltpu.VMEM((2000,), jnp.int32),
            pltpu.VMEM((2000,), jnp.int32),
            pltpu.VMEM((80,), jnp.int32),
            pltpu.VMEM((80,), jnp.int32),
            pltpu.VMEM((80,), jnp.int32),
            pltpu.VMEM((80,), jnp.int32),
            pltpu.VMEM((80,), jnp.int32),
            pltpu.VMEM((80,), jnp.float32),
            pltpu.VMEM((80, _H1), jnp.float32),
            pltpu.VMEM((80,), jnp.int32),
            pltpu.VMEM((80,), jnp.int32),
            pltpu.VMEM((80,), jnp.int32),
            pltpu.VMEM((80,), jnp.int32),
            pltpu.VMEM((80,), jnp.int32),
            pltpu.VMEM((80,), jnp.float32),
            pltpu.VMEM((80, _H1), jnp.float32),
            pltpu.SemaphoreType.DMA,
            pltpu.SemaphoreType.DMA,
        ],
    )(_sc_rgcn_body)
    return kfn(srcv, dstv, etv, tbl)


def _sc_score_body(dstv, srcv, qlo, qhi, klo, khi, sco, pmax,
                   dstbA, srcbA, dstbB, srcbB,
                   qdlA, qdhA, kslA, kshA, qdlB, qdhB, kslB, kshB,
                   scv, mbuf, semA, semB):
    c = lax.axis_index("c")
    s = lax.axis_index("s")
    w = c * _NS + s
    e0 = w * 10000
    nch = 125
    neg = jnp.full((16,), -3.0e38, jnp.float32)
    i16 = lax.iota(jnp.int32, 16)

    bufs = ((dstbA, srcbA, qdlA, qdhA, kslA, kshA, semA),
            (dstbB, srcbB, qdlB, qdhB, kslB, kshB, semB))

    def _start(cix, bi):
        dstb, srcb, qdl, qdh, ksl, ksh, sem = bufs[bi]
        b0 = e0 + cix * 80
        pltpu.sync_copy(dstv.at[pl.ds(b0, 80)], dstb)
        pltpu.sync_copy(srcv.at[pl.ds(b0, 80)], srcb)
        pltpu.async_copy(qlo.at[dstb], qdl, sem)
        pltpu.async_copy(qhi.at[dstb], qdh, sem)
        pltpu.async_copy(klo.at[srcb], ksl, sem)
        pltpu.async_copy(khi.at[srcb], ksh, sem)

    def _wait(bi):
        dstb, srcb, qdl, qdh, ksl, ksh, sem = bufs[bi]
        pltpu.make_async_copy(qlo.at[dstb], qdl, sem).wait()
        pltpu.make_async_copy(qhi.at[dstb], qdh, sem).wait()
        pltpu.make_async_copy(klo.at[srcb], ksl, sem).wait()
        pltpu.make_async_copy(khi.at[srcb], ksh, sem).wait()

    def _compute(cix, bi, carry):
        _, _, qdl, qdh, ksl, ksh, _ = bufs[bi]
        b0 = e0 + cix * 80

        lt8 = i16 < 8
        b4 = (i16 & 4) == 0

        def _sub(t, carry2):
            svec = [jnp.zeros((16,), jnp.float32) for _ in range(4)]
            for jj in range(16):
                e = t * 16 + jj
                lane = i16 == jj
                ps = []
                for h, (qref, kref) in enumerate(
                        ((qdl, ksl), (qdl, ksl), (qdh, ksh), (qdh, ksh))):
                    base = (h % 2) * 64
                    p = jnp.zeros((16,), jnp.float32)
                    for v in range(4):
                        cs = pl.ds(base + v * 16, 16)
                        p = p + qref[e, cs] * kref[e, cs]
                    ps.append(p + p[i16 ^ 8])
                m01 = jnp.where(lt8, ps[0], ps[1])
                m23 = jnp.where(lt8, ps[2], ps[3])
                e1 = m01 + m01[i16 ^ 4]
                f1 = m23 + m23[i16 ^ 4]
                n = jnp.where(b4, e1, f1)
                n = n + n[i16 ^ 2]
                n = (n + n[i16 ^ 1]) * 0.125
                for h, g in enumerate((0, 8, 4, 12)):
                    svec[h] = jnp.where(lane, jnp.broadcast_to(n[g], (16,)),
                                        svec[h])
            sl = pl.ds(t * 16, 16)
            out2 = []
            for h in range(4):
                scv[h, sl] = svec[h]
                out2.append(jnp.maximum(carry2[h], svec[h]))
            return tuple(out2)
        carry = lax.fori_loop(0, 5, _sub, carry)
        for h in range(4):
            pltpu.sync_copy(scv.at[h], sco.at[pl.ds(h * _E + b0, 80)])
        return carry

    _start(0, 0)

    def _pair(i, carry):
        _wait(0)
        _start(2 * i + 1, 1)
        carry = _compute(2 * i, 0, carry)
        _wait(1)
        _start(2 * i + 2, 0)
        carry = _compute(2 * i + 1, 1, carry)
        return carry

    carry = lax.fori_loop(0, (nch - 1) // 2, _pair, (neg, neg, neg, neg))
    _wait(0)
    m0, m1, m2, m3 = _compute(nch - 1, 0, carry)
    mbuf[...] = jnp.maximum(jnp.maximum(m0, m1), jnp.maximum(m2, m3))
    pltpu.sync_copy(mbuf, pmax.at[pl.ds(w * 16, 16)])


def _sc_score(dstv, srcv, qlo, qhi, klo, khi):
    mesh = plsc.VectorSubcoreMesh(core_axis_name="c", subcore_axis_name="s")
    row = lambda: pltpu.VMEM((80, 128), jnp.float32)
    idx = lambda: pltpu.VMEM((80,), jnp.int32)
    kfn = functools.partial(
        pl.kernel,
        out_type=(jax.ShapeDtypeStruct((4 * _E,), jnp.float32),
                  jax.ShapeDtypeStruct((_NC * _NS * 16,), jnp.float32)),
        mesh=mesh,
        scratch_types=[
            idx(), idx(), idx(), idx(),
            row(), row(), row(), row(), row(), row(), row(), row(),
            pltpu.VMEM((4, 80), jnp.float32),
            pltpu.VMEM((16,), jnp.float32),
            pltpu.SemaphoreType.DMA,
            pltpu.SemaphoreType.DMA,
        ],
    )(_sc_score_body)
    return kfn(dstv, srcv, qlo, qhi, klo, khi)


def _sc_attn_body(dstv, srcv, v2, sco, pmax, nump, denp,
                  acc_sp, den_sp, zb2, zbd,
                  dstbA, srcbA, idxvA, idxdaA, idxdbA, saA, sbA, vbA,
                  dstbB, srcbB, idxvB, idxdaB, idxdbB, saB, sbB, vbB,
                  wab, wbb, rowsb, mxv, semA, semB):
    c = lax.axis_index("c")
    s = lax.axis_index("s")

    pltpu.sync_copy(pmax, mxv)
    m = mxv[pl.ds(0, 16)]
    for r in range(1, 32):
        m = jnp.maximum(m, mxv[pl.ds(r * 16, 16)])
    i16g = lax.iota(jnp.int32, 16)
    for st in (8, 4, 2, 1):
        m = jnp.maximum(m, m[i16g ^ st])
    gmax = m

    _fill(zb2, 64, 0.0)
    _fill(zbd, 157, 0.0)
    _zero_rows(zb2, acc_sp, s)
    pltpu.sync_copy(zbd, den_sp.at[pl.ds(s * 2512, 2512)])
    plsc.subcore_barrier()

    bufs = ((dstbA, srcbA, idxvA, idxdaA, idxdbA, saA, sbA, vbA, semA),
            (dstbB, srcbB, idxvB, idxdaB, idxdbB, saB, sbB, vbB, semB))

    def _start(cix, bi):
        dstb, srcb, idxv, idxda, idxdb, sa, sb, vb, sem = bufs[bi]
        b0 = s * 20000 + cix * 80
        pltpu.sync_copy(dstv.at[pl.ds(b0, 80)], dstb)
        pltpu.sync_copy(srcv.at[pl.ds(b0, 80)], srcb)

        def _ix(i, _):
            sl = pl.ds(i * 16, 16)
            idxv[sl] = srcb[sl] + c * _N
            idxda[sl] = dstb[sl] * 4 + 2 * c
            idxdb[sl] = dstb[sl] * 4 + (2 * c + 1)
            return 0
        lax.fori_loop(0, 5, _ix, 0)
        pltpu.async_copy(v2.at[idxv], vb, sem)
        pltpu.sync_copy(
            sco.at[pl.ds(pl.multiple_of(2 * c * _E + b0, 8), 80)], sa)
        pltpu.sync_copy(
            sco.at[pl.ds(pl.multiple_of((2 * c + 1) * _E + b0, 8), 80)], sb)

    def _wait(bi):
        _, _, idxv, _, _, _, _, vb, sem = bufs[bi]
        pltpu.make_async_copy(v2.at[idxv], vb, sem).wait()

    def _compute(bi):
        dstb, srcb, idxv, idxda, idxdb, sa, sb, vb, sem = bufs[bi]

        def _rows(i, _):
            sl = pl.ds(i * 16, 16)
            wa16 = jnp.exp(sa[sl] - gmax)
            wb16 = jnp.exp(sb[sl] - gmax)
            wab[sl] = wa16
            wbb[sl] = wb16
            for jj in range(16):
                e = i * 16 + jj
                wav = _splat(wa16, jj)
                wbv = _splat(wb16, jj)
                for vv in range(4):
                    cs = pl.ds(vv * 16, 16)
                    rowsb[e, cs] = vb[e, cs] * wav
                for vv in range(4, 8):
                    cs = pl.ds(vv * 16, 16)
                    rowsb[e, cs] = vb[e, cs] * wbv
            return 0
        lax.fori_loop(0, 5, _rows, 0)

        pltpu.sync_copy(rowsb, acc_sp.at[dstb], add=True)
        pltpu.sync_copy(wab, den_sp.at[idxda], add=True)
        pltpu.sync_copy(wbb, den_sp.at[idxdb], add=True)

    _start(0, 0)

    def _pair(i, _):
        _wait(0)
        _start(2 * i + 1, 1)
        _compute(0)
        _wait(1)
        _start(2 * i + 2, 0)
        _compute(1)
        return 0
    lax.fori_loop(0, 124, _pair, 0)
    _wait(0)
    _start(249, 1)
    _compute(0)
    _wait(1)
    _compute(1)
    plsc.subcore_barrier()

    _tile_rows(lambda r0, nr: pltpu.sync_copy(
        acc_sp.at[pl.ds(r0, nr)], nump.at[c, pl.ds(r0, nr)]), s)
    pltpu.sync_copy(den_sp.at[pl.ds(s * 2512, 2512)], zbd)
    pltpu.sync_copy(zbd, denp.at[pl.ds(c * _D4 + s * 2512, 2512)])


def _sc_attn(dstv, srcv, v2, sco, pmax):
    mesh = plsc.VectorSubcoreMesh(core_axis_name="c", subcore_axis_name="s")
    idx = lambda: pltpu.VMEM((80,), jnp.int32)
    f80 = lambda: pltpu.VMEM((80,), jnp.float32)
    row = lambda: pltpu.VMEM((80, _H1), jnp.float32)
    kfn = functools.partial(
        pl.kernel,
        out_type=(jax.ShapeDtypeStruct((_NC, _N, _H1), jnp.float32),
                  jax.ShapeDtypeStruct((_NC * _D4,), jnp.float32)),
        mesh=mesh,
        scratch_types=[
            pltpu.VMEM_SHARED((_N, _H1), jnp.float32),
            pltpu.VMEM_SHARED((_D4,), jnp.float32),
            pltpu.VMEM((8, _H1), jnp.float32),
            pltpu.VMEM((2512,), jnp.float32),
            idx(), idx(), idx(), idx(), idx(), f80(), f80(), row(),
            idx(), idx(), idx(), idx(), idx(), f80(), f80(), row(),
            f80(),
            f80(),
            row(),
            pltpu.VMEM((512,), jnp.float32),
            pltpu.SemaphoreType.DMA,
            pltpu.SemaphoreType.DMA,
        ],
    )(_sc_attn_body)
    return kfn(dstv, srcv, v2, sco, pmax)


def _tc_head_body(n0_ref, n1_ref, d0_ref, d1_ref, sk_ref, bsk_ref,
                  h_ref, st_ref):
    i = pl.program_id(0)
    den = d0_ref[...] + d1_ref[...] + 1e-16
    parts = []
    for h in range(4):
        nref = n0_ref if h < 2 else n1_ref
        col = (h % 2) * 64
        parts.append(nref[:, col:col + 64] / den[:, h:h + 1])
    h_val = jnp.concatenate(parts, axis=1) + sk_ref[...] + bsk_ref[...]
    h_ref[...] = h_val

    @pl.when(i == 0)
    def _():
        st_ref[...] = jnp.zeros_like(st_ref)
    st_ref[0:1, :] += jnp.sum(h_val, axis=0, keepdims=True)
    st_ref[1:2, :] += jnp.sum(h_val * h_val, axis=0, keepdims=True)


def _tc_head(n0, n1, d0, d1, sk, bsk, bn=2000):
    n = n0.shape[0]
    return pl.pallas_call(
        _tc_head_body,
        grid=(n // bn,),
        in_specs=[pl.BlockSpec((bn, 128), lambda i: (i, 0)),
                  pl.BlockSpec((bn, 128), lambda i: (i, 0)),
                  pl.BlockSpec((bn, 4), lambda i: (i, 0)),
                  pl.BlockSpec((bn, 4), lambda i: (i, 0)),
                  pl.BlockSpec((bn, 256), lambda i: (i, 0)),
                  pl.BlockSpec((1, 256), lambda i: (0, 0))],
        out_specs=(pl.BlockSpec((bn, 256), lambda i: (i, 0)),
                   pl.BlockSpec((8, 256), lambda i: (0, 0))),
        out_shape=(jax.ShapeDtypeStruct((n, 256), jnp.float32),
                   jax.ShapeDtypeStruct((8, 256), jnp.float32)),
    )(n0, n1, d0, d1, sk, bsk)


def _tc_bn_body(h_ref, st_ref, g_ref, b_ref, o_ref):
    h = h_ref[...]
    n = jnp.float32(_N)
    mean = st_ref[0:1, :] / n
    var = st_ref[1:2, :] / n - mean * mean
    y = (h - mean) / jnp.sqrt(var + 1e-5) * g_ref[...] + b_ref[...]
    o_ref[...] = jnp.where(y > 0, y, 0.01 * y)


def _tc_bn(h, st, g, b, bn=2000):
    n = h.shape[0]
    return pl.pallas_call(
        _tc_bn_body,
        grid=(n // bn,),
        in_specs=[pl.BlockSpec((bn, 256), lambda i: (i, 0)),
                  pl.BlockSpec((8, 256), lambda i: (0, 0)),
                  pl.BlockSpec((1, 256), lambda i: (0, 0)),
                  pl.BlockSpec((1, 256), lambda i: (0, 0))],
        out_specs=pl.BlockSpec((bn, 256), lambda i: (i, 0)),
        out_shape=jax.ShapeDtypeStruct((n, 256), jnp.float32),
    )(h, st, g, b)


def kernel(node_features, node_type, edge_index, edge_type, W_rel, W_root,
           b_rgcn, Wq, bq, Wk, bk, Wv, bv, Wskip, bskip, gamma, beta):
    del node_type
    srcv = edge_index[0].astype(jnp.int32)
    dstv = edge_index[1].astype(jnp.int32)
    etv = edge_type.astype(jnp.int32)

    w2 = W_rel.transpose(1, 0, 2).reshape(_G, _R * _H1)
    wcat1 = jnp.concatenate([w2, W_root], axis=1)
    mm1 = _tc_matmul(node_features, wcat1)
    tbl = mm1[:, :_R * _H1].reshape(_N * _R, _H1)
    xr = mm1[:, _R * _H1:]

    aggp = _sc_rgcn(srcv, dstv, etv, tbl)

    wcat2 = jnp.concatenate([Wq, Wk, Wv, Wskip], axis=1)
    bcat2 = jnp.concatenate([bq, bk, bv, bskip]).reshape(1, 1024)
    qkvs = _tc_fuse(aggp[0], aggp[1], xr, b_rgcn.reshape(1, _H1),
                    wcat2, bcat2)
    qlo, qhi = qkvs[:, 0:128], qkvs[:, 128:256]
    klo, khi = qkvs[:, 256:384], qkvs[:, 384:512]
    v2 = jnp.concatenate([qkvs[:, 512:640], qkvs[:, 640:768]], axis=0)
    sk = qkvs[:, 768:1024]

    sco, pmax = _sc_score(dstv, srcv, qlo, qhi, klo, khi)

    nump, denp = _sc_attn(dstv, srcv, v2, sco, pmax)

    d0 = denp[:_N * 4].reshape(_N, 4)
    d1 = denp[_D4:_D4 + _N * 4].reshape(_N, 4)
    h, st = _tc_head(nump[0], nump[1], d0, d1, sk, bskip.reshape(1, 256))
    out = _tc_bn(h, st, gamma.reshape(1, 256), beta.reshape(1, 256))
    return out


# speedup vs baseline: 1.1627x; 1.1627x over previous
import functools
import numpy as np
import jax
import jax.numpy as jnp
from jax import lax
from jax.experimental import pallas as pl
from jax.experimental.pallas import tpu as pltpu, tpu_sc as plsc

_N = 10000
_E = 320000
_G = 128
_H1 = 128
_H2 = 64
_R = 8
_HEADS = 4
_NC = 2
_NS = 16

_D4 = 40192


def _splat(vec16, j):
    return jnp.broadcast_to(vec16[j], (16,))


def _fill(ref, nvec, value):
    v = jnp.full((16,), value, jnp.float32)
    if len(ref.shape) == 1:
        def _b(i, _):
            ref[pl.ds(i * 16, 16)] = v
            return 0
    else:
        ncol = ref.shape[1] // 16

        def _b(i, _):
            ref[i // ncol, pl.ds((i % ncol) * 16, 16)] = v
            return 0
    lax.fori_loop(0, nvec, _b, 0)


def _zero_rows(zb8, sp_ref, s):
    def _b(j, _):
        pltpu.sync_copy(zb8, sp_ref.at[pl.ds(s * 624 + j * 8, 8)])
        return 0
    lax.fori_loop(0, 78, _b, 0)

    @pl.when(s == _NS - 1)
    def _():
        pltpu.sync_copy(zb8, sp_ref.at[pl.ds(9984, 8)])
        pltpu.sync_copy(zb8, sp_ref.at[pl.ds(9992, 8)])


def _tile_rows(sync_fn, s):
    def _b(j, _):
        sync_fn(s * 624 + j * 208, 208)
        return 0
    lax.fori_loop(0, 3, _b, 0)

    @pl.when(s == _NS - 1)
    def _():
        sync_fn(9984, 16)


def _tc_matmul_body(x_ref, w_ref, o_ref):
    o_ref[...] = jnp.dot(x_ref[...], w_ref[...],
                         preferred_element_type=jnp.float32)


def _tc_matmul(x, w, bn=2000):
    n, kdim = x.shape
    m = w.shape[1]
    return pl.pallas_call(
        _tc_matmul_body,
        grid=(n // bn,),
        in_specs=[pl.BlockSpec((bn, kdim), lambda i: (i, 0)),
                  pl.BlockSpec((kdim, m), lambda i: (0, 0))],
        out_specs=pl.BlockSpec((bn, m), lambda i: (i, 0)),
        out_shape=jax.ShapeDtypeStruct((n, m), jnp.float32),
    )(x, w)


def _tc_fuse_body(a0_ref, a1_ref, xr_ref, b_ref, w_ref, b2_ref, o_ref):
    x2 = a0_ref[...] + a1_ref[...] + xr_ref[...] + b_ref[...]
    o_ref[...] = jnp.dot(x2, w_ref[...],
                         preferred_element_type=jnp.float32) + b2_ref[...]


def _tc_fuse(a0, a1, xr, b, w, b2, bn=2000):
    n, kdim = a0.shape
    m = w.shape[1]
    return pl.pallas_call(
        _tc_fuse_body,
        grid=(n // bn,),
        in_specs=[pl.BlockSpec((bn, kdim), lambda i: (i, 0)),
                  pl.BlockSpec((bn, kdim), lambda i: (i, 0)),
                  pl.BlockSpec((bn, kdim), lambda i: (i, 0)),
                  pl.BlockSpec((1, kdim), lambda i: (0, 0)),
                  pl.BlockSpec((kdim, m), lambda i: (0, 0)),
                  pl.BlockSpec((1, m), lambda i: (0, 0))],
        out_specs=pl.BlockSpec((bn, m), lambda i: (i, 0)),
        out_shape=jax.ShapeDtypeStruct((n, m), jnp.float32),
    )(a0, a1, xr, b, w, b2)


def _sc_rgcn_body(srcv, dstv, etv, tbl, aggp,
                  cnt_sp, agg_sp, onesb, zb2, zbd,
                  dstb, etb, idxc,
                  srcbA, etb2A, dstb2A, idxmA, idxc2A, cntgA, msgA,
                  srcbB, etb2B, dstb2B, idxmB, idxc2B, cntgB, msgB,
                  semA, semB):
    c = lax.axis_index("c")
    s = lax.axis_index("s")

    _fill(onesb, 125, 1.0)
    _fill(zb2, 64, 0.0)
    _fill(zbd, 157, 0.0)

    pltpu.sync_copy(zbd, cnt_sp.at[pl.ds(s * 5008, 2512)])
    pltpu.sync_copy(zbd, cnt_sp.at[pl.ds(s * 5008 + 2496, 2512)])
    _zero_rows(zb2, agg_sp, s)
    plsc.subcore_barrier()

    def _cnt(j, _):
        b0 = s * 20000 + j * 2000
        pltpu.sync_copy(dstv.at[pl.ds(b0, 2000)], dstb)
        pltpu.sync_copy(etv.at[pl.ds(b0, 2000)], etb)

        def _ix(i, _):
            sl = pl.ds(i * 16, 16)
            idxc[sl] = dstb[sl] * _R + etb[sl]
            return 0
        lax.fori_loop(0, 125, _ix, 0)
        pltpu.sync_copy(onesb, cnt_sp.at[idxc], add=True)
        return 0
    lax.fori_loop(0, 10, _cnt, 0)
    plsc.subcore_barrier()

    bufs = ((srcbA, etb2A, dstb2A, idxmA, idxc2A, cntgA, msgA, semA),
            (srcbB, etb2B, dstb2B, idxmB, idxc2B, cntgB, msgB, semB))

    def _start(cix, bi):
        srcb, etb2, dstb2, idxm, idxc2, cntg, msg, sem = bufs[bi]
        b0 = c * 160000 + s * 10000 + cix * 80
        pltpu.sync_copy(srcv.at[pl.ds(b0, 80)], srcb)
        pltpu.sync_copy(etv.at[pl.ds(b0, 80)], etb2)
        pltpu.sync_copy(dstv.at[pl.ds(b0, 80)], dstb2)

        def _ix2(i, _):
            sl = pl.ds(i * 16, 16)
            idxm[sl] = srcb[sl] * _R + etb2[sl]
            idxc2[sl] = dstb2[sl] * _R + etb2[sl]
            return 0
        lax.fori_loop(0, 5, _ix2, 0)
        pltpu.async_copy(tbl.at[idxm], msg, sem)

    def _wait(bi):
        srcb, etb2, dstb2, idxm, idxc2, cntg, msg, sem = bufs[bi]
        pltpu.make_async_copy(tbl.at[idxm], msg, sem).wait()
        pltpu.async_copy(cnt_sp.at[idxc2], cntg, sem).wait()

    def _compute(bi):
        srcb, etb2, dstb2, idxm, idxc2, cntg, msg, sem = bufs[bi]

        def _scale(i, _):
            c16 = cntg[pl.ds(i * 16, 16)]
            n16 = 1.0 / jnp.maximum(c16, 1.0)
            for jj in range(16):
                e = i * 16 + jj
                nv = _splat(n16, jj)
                for vv in range(8):
                    sl = pl.ds(vv * 16, 16)
                    msg[e, sl] = msg[e, sl] * nv
            return 0
        lax.fori_loop(0, 5, _scale, 0)
        pltpu.sync_copy(msg, agg_sp.at[dstb2], add=True)

    _start(0, 0)

    def _pair(i, _):
        _wait(0)
        _start(2 * i + 1, 1)
        _compute(0)
        _wait(1)
        _start(2 * i + 2, 0)
        _compute(1)
        return 0
    lax.fori_loop(0, 62, _pair, 0)
    _wait(0)
    _compute(0)
    plsc.subcore_barrier()

    _tile_rows(lambda r0, nr: pltpu.sync_copy(
        agg_sp.at[pl.ds(r0, nr)], aggp.at[c, pl.ds(r0, nr)]), s)


def _sc_rgcn(srcv, dstv, etv, tbl):
    mesh = plsc.VectorSubcoreMesh(core_axis_name="c", subcore_axis_name="s")
    kfn = functools.partial(
        pl.kernel,
        out_type=jax.ShapeDtypeStruct((_NC, _N, _H1), jnp.float32),
        mesh=mesh,
        scratch_types=[
            pltpu.VMEM_SHARED((80128,), jnp.float32),
            pltpu.VMEM_SHARED((_N, _H1), jnp.float32),
            pltpu.VMEM((2000,), jnp.float32),
            pltpu.VMEM((8, _H1), jnp.float32),
            pltpu.VMEM((2512,), jnp.float32),
            pltpu.VMEM((2000,), jnp.int32),
            pltpu.VMEM((2000,), jnp.int32),
            pltpu.VMEM((2000,), jnp.int32),
            pltpu.VMEM((80,), jnp.int32),
            pltpu.VMEM((80,), jnp.int32),
            pltpu.VMEM((80,), jnp.int32),
            pltpu.VMEM((80,), jnp.int32),
            pltpu.VMEM((80,), jnp.int32),
            pltpu.VMEM((80,), jnp.float32),
            pltpu.VMEM((80, _H1), jnp.float32),
            pltpu.VMEM((80,), jnp.int32),
            pltpu.VMEM((80,), jnp.int32),
            pltpu.VMEM((80,), jnp.int32),
            pltpu.VMEM((80,), jnp.int32),
            pltpu.VMEM((80,), jnp.int32),
            pltpu.VMEM((80,), jnp.float32),
            pltpu.VMEM((80, _H1), jnp.float32),
            pltpu.SemaphoreType.DMA,
            pltpu.SemaphoreType.DMA,
        ],
    )(_sc_rgcn_body)
    return kfn(srcv, dstv, etv, tbl)


def _sc_score_body(dstv, srcv, qlo, qhi, klo, khi, sco, pmax,
                   dstbA, srcbA, dstbB, srcbB,
                   qdlA, qdhA, kslA, kshA, qdlB, qdhB, kslB, kshB,
                   scv, mbuf, semA, semB):
    c = lax.axis_index("c")
    s = lax.axis_index("s")
    w = c * _NS + s
    e0 = w * 10000
    nch = 125
    neg = jnp.full((16,), -3.0e38, jnp.float32)
    i16 = lax.iota(jnp.int32, 16)

    bufs = ((dstbA, srcbA, qdlA, qdhA, kslA, kshA, semA),
            (dstbB, srcbB, qdlB, qdhB, kslB, kshB, semB))

    def _start(cix, bi):
        dstb, srcb, qdl, qdh, ksl, ksh, sem = bufs[bi]
        b0 = e0 + cix * 80
        pltpu.sync_copy(dstv.at[pl.ds(b0, 80)], dstb)
        pltpu.sync_copy(srcv.at[pl.ds(b0, 80)], srcb)
        pltpu.async_copy(qlo.at[dstb], qdl, sem)
        pltpu.async_copy(qhi.at[dstb], qdh, sem)
        pltpu.async_copy(klo.at[srcb], ksl, sem)
        pltpu.async_copy(khi.at[srcb], ksh, sem)

    def _wait(bi):
        dstb, srcb, qdl, qdh, ksl, ksh, sem = bufs[bi]
        pltpu.make_async_copy(qlo.at[dstb], qdl, sem).wait()
        pltpu.make_async_copy(qhi.at[dstb], qdh, sem).wait()
        pltpu.make_async_copy(klo.at[srcb], ksl, sem).wait()
        pltpu.make_async_copy(khi.at[srcb], ksh, sem).wait()

    def _compute(cix, bi, carry):
        _, _, qdl, qdh, ksl, ksh, _ = bufs[bi]
        b0 = e0 + cix * 80

        def _sub(t, carry2):
            svec = [jnp.zeros((16,), jnp.float32) for _ in range(4)]
            for jj in range(16):
                e = t * 16 + jj
                lane = i16 == jj
                for h, (qref, kref) in enumerate(
                        ((qdl, ksl), (qdl, ksl), (qdh, ksh), (qdh, ksh))):
                    base = (h % 2) * 64
                    p = jnp.zeros((16,), jnp.float32)
                    for v in range(4):
                        cs = pl.ds(base + v * 16, 16)
                        p = p + qref[e, cs] * kref[e, cs]
                    for st in (8, 4, 2, 1):
                        p = p + p[i16 ^ st]
                    svec[h] = jnp.where(lane, p * 0.125, svec[h])
            sl = pl.ds(t * 16, 16)
            out2 = []
            for h in range(4):
                scv[h, sl] = svec[h]
                out2.append(jnp.maximum(carry2[h], svec[h]))
            return tuple(out2)
        carry = lax.fori_loop(0, 5, _sub, carry)
        for h in range(4):
            pltpu.sync_copy(scv.at[h], sco.at[pl.ds(h * _E + b0, 80)])
        return carry

    _start(0, 0)

    def _pair(i, carry):
        _wait(0)
        _start(2 * i + 1, 1)
        carry = _compute(2 * i, 0, carry)
        _wait(1)
        _start(2 * i + 2, 0)
        carry = _compute(2 * i + 1, 1, carry)
        return carry

    carry = lax.fori_loop(0, (nch - 1) // 2, _pair, (neg, neg, neg, neg))
    _wait(0)
    m0, m1, m2, m3 = _compute(nch - 1, 0, carry)
    mbuf[...] = jnp.maximum(jnp.maximum(m0, m1), jnp.maximum(m2, m3))
    pltpu.sync_copy(mbuf, pmax.at[pl.ds(w * 16, 16)])


def _sc_score(dstv, srcv, qlo, qhi, klo, khi):
    mesh = plsc.VectorSubcoreMesh(core_axis_name="c", subcore_axis_name="s")
    row = lambda: pltpu.VMEM((80, 128), jnp.float32)
    idx = lambda: pltpu.VMEM((80,), jnp.int32)
    kfn = functools.partial(
        pl.kernel,
        out_type=(jax.ShapeDtypeStruct((4 * _E,), jnp.float32),
                  jax.ShapeDtypeStruct((_NC * _NS * 16,), jnp.float32)),
        mesh=mesh,
        scratch_types=[
            idx(), idx(), idx(), idx(),
            row(), row(), row(), row(), row(), row(), row(), row(),
            pltpu.VMEM((4, 80), jnp.float32),
            pltpu.VMEM((16,), jnp.float32),
            pltpu.SemaphoreType.DMA,
            pltpu.SemaphoreType.DMA,
        ],
    )(_sc_score_body)
    return kfn(dstv, srcv, qlo, qhi, klo, khi)


def _sc_attn_body(dstv, srcv, v2, sco, pmax, nump, denp,
                  acc_sp, den_sp, zb2, zbd,
                  dstbA, srcbA, idxvA, idxdaA, idxdbA, saA, sbA, vbA,
                  dstbB, srcbB, idxvB, idxdaB, idxdbB, saB, sbB, vbB,
                  wab, wbb, rowsb, mxv, semA, semB):
    c = lax.axis_index("c")
    s = lax.axis_index("s")

    pltpu.sync_copy(pmax, mxv)
    m = mxv[pl.ds(0, 16)]
    for r in range(1, 32):
        m = jnp.maximum(m, mxv[pl.ds(r * 16, 16)])
    i16g = lax.iota(jnp.int32, 16)
    for st in (8, 4, 2, 1):
        m = jnp.maximum(m, m[i16g ^ st])
    gmax = m

    _fill(zb2, 64, 0.0)
    _fill(zbd, 157, 0.0)
    _zero_rows(zb2, acc_sp, s)
    pltpu.sync_copy(zbd, den_sp.at[pl.ds(s * 2512, 2512)])
    plsc.subcore_barrier()

    bufs = ((dstbA, srcbA, idxvA, idxdaA, idxdbA, saA, sbA, vbA, semA),
            (dstbB, srcbB, idxvB, idxdaB, idxdbB, saB, sbB, vbB, semB))

    def _start(cix, bi):
        dstb, srcb, idxv, idxda, idxdb, sa, sb, vb, sem = bufs[bi]
        b0 = s * 20000 + cix * 80
        pltpu.sync_copy(dstv.at[pl.ds(b0, 80)], dstb)
        pltpu.sync_copy(srcv.at[pl.ds(b0, 80)], srcb)

        def _ix(i, _):
            sl = pl.ds(i * 16, 16)
            idxv[sl] = srcb[sl] + c * _N
            idxda[sl] = dstb[sl] * 4 + 2 * c
            idxdb[sl] = dstb[sl] * 4 + (2 * c + 1)
            return 0
        lax.fori_loop(0, 5, _ix, 0)
        pltpu.async_copy(v2.at[idxv], vb, sem)
        pltpu.sync_copy(
            sco.at[pl.ds(pl.multiple_of(2 * c * _E + b0, 8), 80)], sa)
        pltpu.sync_copy(
            sco.at[pl.ds(pl.multiple_of((2 * c + 1) * _E + b0, 8), 80)], sb)

    def _wait(bi):
        _, _, idxv, _, _, _, _, vb, sem = bufs[bi]
        pltpu.make_async_copy(v2.at[idxv], vb, sem).wait()

    def _compute(bi):
        dstb, srcb, idxv, idxda, idxdb, sa, sb, vb, sem = bufs[bi]

        def _rows(i, _):
            sl = pl.ds(i * 16, 16)
            wa16 = jnp.exp(sa[sl] - gmax)
            wb16 = jnp.exp(sb[sl] - gmax)
            wab[sl] = wa16
            wbb[sl] = wb16
            for jj in range(16):
                e = i * 16 + jj
                wav = _splat(wa16, jj)
                wbv = _splat(wb16, jj)
                for vv in range(4):
                    cs = pl.ds(vv * 16, 16)
                    rowsb[e, cs] = vb[e, cs] * wav
                for vv in range(4, 8):
                    cs = pl.ds(vv * 16, 16)
                    rowsb[e, cs] = vb[e, cs] * wbv
            return 0
        lax.fori_loop(0, 5, _rows, 0)

        pltpu.sync_copy(rowsb, acc_sp.at[dstb], add=True)
        pltpu.sync_copy(wab, den_sp.at[idxda], add=True)
        pltpu.sync_copy(wbb, den_sp.at[idxdb], add=True)

    _start(0, 0)

    def _pair(i, _):
        _wait(0)
        _start(2 * i + 1, 1)
        _compute(0)
        _wait(1)
        _start(2 * i + 2, 0)
        _compute(1)
        return 0
    lax.fori_loop(0, 124, _pair, 0)
    _wait(0)
    _start(249, 1)
    _compute(0)
    _wait(1)
    _compute(1)
    plsc.subcore_barrier()

    _tile_rows(lambda r0, nr: pltpu.sync_copy(
        acc_sp.at[pl.ds(r0, nr)], nump.at[c, pl.ds(r0, nr)]), s)
    pltpu.sync_copy(den_sp.at[pl.ds(s * 2512, 2512)], zbd)
    pltpu.sync_copy(zbd, denp.at[pl.ds(c * _D4 + s * 2512, 2512)])


def _sc_attn(dstv, srcv, v2, sco, pmax):
    mesh = plsc.VectorSubcoreMesh(core_axis_name="c", subcore_axis_name="s")
    idx = lambda: pltpu.VMEM((80,), jnp.int32)
    f80 = lambda: pltpu.VMEM((80,), jnp.float32)
    row = lambda: pltpu.VMEM((80, _H1), jnp.float32)
    kfn = functools.partial(
        pl.kernel,
        out_type=(jax.ShapeDtypeStruct((_NC, _N, _H1), jnp.float32),
                  jax.ShapeDtypeStruct((_NC * _D4,), jnp.float32)),
        mesh=mesh,
        scratch_types=[
            pltpu.VMEM_SHARED((_N, _H1), jnp.float32),
            pltpu.VMEM_SHARED((_D4,), jnp.float32),
            pltpu.VMEM((8, _H1), jnp.float32),
            pltpu.VMEM((2512,), jnp.float32),
            idx(), idx(), idx(), idx(), idx(), f80(), f80(), row(),
            idx(), idx(), idx(), idx(), idx(), f80(), f80(), row(),
            f80(),
            f80(),
            row(),
            pltpu.VMEM((512,), jnp.float32),
            pltpu.SemaphoreType.DMA,
            pltpu.SemaphoreType.DMA,
        ],
    )(_sc_attn_body)
    return kfn(dstv, srcv, v2, sco, pmax)


def _tc_head_body(n0_ref, n1_ref, d0_ref, d1_ref, sk_ref, bsk_ref,
                  h_ref, st_ref):
    i = pl.program_id(0)
    den = d0_ref[...] + d1_ref[...] + 1e-16
    parts = []
    for h in range(4):
        nref = n0_ref if h < 2 else n1_ref
        col = (h % 2) * 64
        parts.append(nref[:, col:col + 64] / den[:, h:h + 1])
    h_val = jnp.concatenate(parts, axis=1) + sk_ref[...] + bsk_ref[...]
    h_ref[...] = h_val

    @pl.when(i == 0)
    def _():
        st_ref[...] = jnp.zeros_like(st_ref)
    st_ref[0:1, :] += jnp.sum(h_val, axis=0, keepdims=True)
    st_ref[1:2, :] += jnp.sum(h_val * h_val, axis=0, keepdims=True)


def _tc_head(n0, n1, d0, d1, sk, bsk, bn=2000):
    n = n0.shape[0]
    return pl.pallas_call(
        _tc_head_body,
        grid=(n // bn,),
        in_specs=[pl.BlockSpec((bn, 128), lambda i: (i, 0)),
                  pl.BlockSpec((bn, 128), lambda i: (i, 0)),
                  pl.BlockSpec((bn, 4), lambda i: (i, 0)),
                  pl.BlockSpec((bn, 4), lambda i: (i, 0)),
                  pl.BlockSpec((bn, 256), lambda i: (i, 0)),
                  pl.BlockSpec((1, 256), lambda i: (0, 0))],
        out_specs=(pl.BlockSpec((bn, 256), lambda i: (i, 0)),
                   pl.BlockSpec((8, 256), lambda i: (0, 0))),
        out_shape=(jax.ShapeDtypeStruct((n, 256), jnp.float32),
                   jax.ShapeDtypeStruct((8, 256), jnp.float32)),
    )(n0, n1, d0, d1, sk, bsk)


def _tc_bn_body(h_ref, st_ref, g_ref, b_ref, o_ref):
    h = h_ref[...]
    n = jnp.float32(_N)
    mean = st_ref[0:1, :] / n
    var = st_ref[1:2, :] / n - mean * mean
    y = (h - mean) / jnp.sqrt(var + 1e-5) * g_ref[...] + b_ref[...]
    o_ref[...] = jnp.where(y > 0, y, 0.01 * y)


def _tc_bn(h, st, g, b, bn=2000):
    n = h.shape[0]
    return pl.pallas_call(
        _tc_bn_body,
        grid=(n // bn,),
        in_specs=[pl.BlockSpec((bn, 256), lambda i: (i, 0)),
                  pl.BlockSpec((8, 256), lambda i: (0, 0)),
                  pl.BlockSpec((1, 256), lambda i: (0, 0)),
                  pl.BlockSpec((1, 256), lambda i: (0, 0))],
        out_specs=pl.BlockSpec((bn, 256), lambda i: (i, 0)),
        out_shape=jax.ShapeDtypeStruct((n, 256), jnp.float32),
    )(h, st, g, b)


def kernel(node_features, node_type, edge_index, edge_type, W_rel, W_root,
           b_rgcn, Wq, bq, Wk, bk, Wv, bv, Wskip, bskip, gamma, beta):
    del node_type
    srcv = edge_index[0].astype(jnp.int32)
    dstv = edge_index[1].astype(jnp.int32)
    etv = edge_type.astype(jnp.int32)

    w2 = W_rel.transpose(1, 0, 2).reshape(_G, _R * _H1)
    wcat1 = jnp.concatenate([w2, W_root], axis=1)
    mm1 = _tc_matmul(node_features, wcat1)
    tbl = mm1[:, :_R * _H1].reshape(_N * _R, _H1)
    xr = mm1[:, _R * _H1:]

    aggp = _sc_rgcn(srcv, dstv, etv, tbl)

    wcat2 = jnp.concatenate([Wq, Wk, Wv, Wskip], axis=1)
    bcat2 = jnp.concatenate([bq, bk, bv, bskip]).reshape(1, 1024)
    qkvs = _tc_fuse(aggp[0], aggp[1], xr, b_rgcn.reshape(1, _H1),
                    wcat2, bcat2)
    qlo, qhi = qkvs[:, 0:128], qkvs[:, 128:256]
    klo, khi = qkvs[:, 256:384], qkvs[:, 384:512]
    v2 = jnp.concatenate([qkvs[:, 512:640], qkvs[:, 640:768]], axis=0)
    sk = qkvs[:, 768:1024]

    sco, pmax = _sc_score(dstv, srcv, qlo, qhi, klo, khi)

    nump, denp = _sc_attn(dstv, srcv, v2, sco, pmax)

    d0 = denp[:_N * 4].reshape(_N, 4)
    d1 = denp[_D4:_D4 + _N * 4].reshape(_N, 4)
    h, st = _tc_head(nump[0], nump[1], d0, d1, sk, bskip.reshape(1, 256))
    out = _tc_bn(h, st, gamma.reshape(1, 256), beta.reshape(1, 256))
    return out


# speedup vs baseline: 1.1709x; 1.0070x over previous
import functools
import numpy as np
import jax
import jax.numpy as jnp
from jax import lax
from jax.experimental import pallas as pl
from jax.experimental.pallas import tpu as pltpu, tpu_sc as plsc

_N = 10000
_E = 320000
_G = 128
_H1 = 128
_H2 = 64
_R = 8
_HEADS = 4
_NC = 2
_NS = 16

_D4 = 40192


def _splat(vec16, j):
    return jnp.broadcast_to(vec16[j], (16,))


def _fill(ref, nvec, value):
    v = jnp.full((16,), value, jnp.float32)
    if len(ref.shape) == 1:
        def _b(i, _):
            ref[pl.ds(i * 16, 16)] = v
            return 0
    else:
        ncol = ref.shape[1] // 16

        def _b(i, _):
            ref[i // ncol, pl.ds((i % ncol) * 16, 16)] = v
            return 0
    lax.fori_loop(0, nvec, _b, 0)


def _zero_rows(zb8, sp_ref, s):
    def _b(j, _):
        pltpu.sync_copy(zb8, sp_ref.at[pl.ds(s * 624 + j * 8, 8)])
        return 0
    lax.fori_loop(0, 78, _b, 0)

    @pl.when(s == _NS - 1)
    def _():
        pltpu.sync_copy(zb8, sp_ref.at[pl.ds(9984, 8)])
        pltpu.sync_copy(zb8, sp_ref.at[pl.ds(9992, 8)])


def _tile_rows(sync_fn, s):
    def _b(j, _):
        sync_fn(s * 624 + j * 208, 208)
        return 0
    lax.fori_loop(0, 3, _b, 0)

    @pl.when(s == _NS - 1)
    def _():
        sync_fn(9984, 16)


def _tc_matmul_body(x_ref, w_ref, o_ref):
    o_ref[...] = jnp.dot(x_ref[...], w_ref[...],
                         preferred_element_type=jnp.float32)


def _tc_matmul(x, w, bn=2000):
    n, kdim = x.shape
    m = w.shape[1]
    return pl.pallas_call(
        _tc_matmul_body,
        grid=(n // bn,),
        in_specs=[pl.BlockSpec((bn, kdim), lambda i: (i, 0)),
                  pl.BlockSpec((kdim, m), lambda i: (0, 0))],
        out_specs=pl.BlockSpec((bn, m), lambda i: (i, 0)),
        out_shape=jax.ShapeDtypeStruct((n, m), jnp.float32),
    )(x, w)


def _tc_fuse_body(a0_ref, a1_ref, xr_ref, b_ref, w_ref, b2_ref, o_ref):
    x2 = a0_ref[...] + a1_ref[...] + xr_ref[...] + b_ref[...]
    o_ref[...] = jnp.dot(x2, w_ref[...],
                         preferred_element_type=jnp.float32) + b2_ref[...]


def _tc_fuse(a0, a1, xr, b, w, b2, bn=2000):
    n, kdim = a0.shape
    m = w.shape[1]
    return pl.pallas_call(
        _tc_fuse_body,
        grid=(n // bn,),
        in_specs=[pl.BlockSpec((bn, kdim), lambda i: (i, 0)),
                  pl.BlockSpec((bn, kdim), lambda i: (i, 0)),
                  pl.BlockSpec((bn, kdim), lambda i: (i, 0)),
                  pl.BlockSpec((1, kdim), lambda i: (0, 0)),
                  pl.BlockSpec((kdim, m), lambda i: (0, 0)),
                  pl.BlockSpec((1, m), lambda i: (0, 0))],
        out_specs=pl.BlockSpec((bn, m), lambda i: (i, 0)),
        out_shape=jax.ShapeDtypeStruct((n, m), jnp.float32),
    )(a0, a1, xr, b, w, b2)


def _sc_rgcn_body(srcv, dstv, etv, tbl, aggp,
                  cnt_sp, agg_sp, onesb, zb2, zbd,
                  dstb, etb, idxc,
                  srcbA, etb2A, dstb2A, idxmA, idxc2A, cntgA, msgA,
                  srcbB, etb2B, dstb2B, idxmB, idxc2B, cntgB, msgB,
                  semA, semB):
    c = lax.axis_index("c")
    s = lax.axis_index("s")

    _fill(onesb, 125, 1.0)
    _fill(zb2, 64, 0.0)
    _fill(zbd, 157, 0.0)

    pltpu.sync_copy(zbd, cnt_sp.at[pl.ds(s * 5008, 2512)])
    pltpu.sync_copy(zbd, cnt_sp.at[pl.ds(s * 5008 + 2496, 2512)])
    _zero_rows(zb2, agg_sp, s)
    plsc.subcore_barrier()

    def _cnt(j, _):
        b0 = s * 20000 + j * 2000
        pltpu.sync_copy(dstv.at[pl.ds(b0, 2000)], dstb)
        pltpu.sync_copy(etv.at[pl.ds(b0, 2000)], etb)

        def _ix(i, _):
            sl = pl.ds(i * 16, 16)
            idxc[sl] = dstb[sl] * _R + etb[sl]
            return 0
        lax.fori_loop(0, 125, _ix, 0)
        pltpu.sync_copy(onesb, cnt_sp.at[idxc], add=True)
        return 0
    lax.fori_loop(0, 10, _cnt, 0)
    plsc.subcore_barrier()

    bufs = ((srcbA, etb2A, dstb2A, idxmA, idxc2A, cntgA, msgA, semA),
            (srcbB, etb2B, dstb2B, idxmB, idxc2B, cntgB, msgB, semB))

    def _start(cix, bi):
        srcb, etb2, dstb2, idxm, idxc2, cntg, msg, sem = bufs[bi]
        b0 = c * 160000 + s * 10000 + cix * 80
        pltpu.sync_copy(srcv.at[pl.ds(b0, 80)], srcb)
        pltpu.sync_copy(etv.at[pl.ds(b0, 80)], etb2)
        pltpu.sync_copy(dstv.at[pl.ds(b0, 80)], dstb2)

        def _ix2(i, _):
            sl = pl.ds(i * 16, 16)
            idxm[sl] = srcb[sl] * _R + etb2[sl]
            idxc2[sl] = dstb2[sl] * _R + etb2[sl]
            return 0
        lax.fori_loop(0, 5, _ix2, 0)
        pltpu.async_copy(tbl.at[idxm], msg, sem)

    def _wait(bi):
        srcb, etb2, dstb2, idxm, idxc2, cntg, msg, sem = bufs[bi]
        pltpu.make_async_copy(tbl.at[idxm], msg, sem).wait()
        pltpu.async_copy(cnt_sp.at[idxc2], cntg, sem).wait()

    def _compute(bi):
        srcb, etb2, dstb2, idxm, idxc2, cntg, msg, sem = bufs[bi]

        def _scale(i, _):
            c16 = cntg[pl.ds(i * 16, 16)]
            n16 = 1.0 / jnp.maximum(c16, 1.0)
            for jj in range(16):
                e = i * 16 + jj
                nv = _splat(n16, jj)
                for vv in range(8):
                    sl = pl.ds(vv * 16, 16)
                    msg[e, sl] = msg[e, sl] * nv
            return 0
        lax.fori_loop(0, 5, _scale, 0)
        pltpu.sync_copy(msg, agg_sp.at[dstb2], add=True)

    _start(0, 0)

    def _pair(i, _):
        _wait(0)
        _start(2 * i + 1, 1)
        _compute(0)
        _wait(1)
        _start(2 * i + 2, 0)
        _compute(1)
        return 0
    lax.fori_loop(0, 62, _pair, 0)
    _wait(0)
    _compute(0)
    plsc.subcore_barrier()

    _tile_rows(lambda r0, nr: pltpu.sync_copy(
        agg_sp.at[pl.ds(r0, nr)], aggp.at[c, pl.ds(r0, nr)]), s)


def _sc_rgcn(srcv, dstv, etv, tbl):
    mesh = plsc.VectorSubcoreMesh(core_axis_name="c", subcore_axis_name="s")
    kfn = functools.partial(
        pl.kernel,
        out_type=jax.ShapeDtypeStruct((_NC, _N, _H1), jnp.float32),
        mesh=mesh,
        scratch_types=[
            pltpu.VMEM_SHARED((80128,), jnp.float32),
            pltpu.VMEM_SHARED((_N, _H1), jnp.float32),
            pltpu.VMEM((2000,), jnp.float32),
            pltpu.VMEM((8, _H1), jnp.float32),
            pltpu.VMEM((2512,), jnp.float32),
            pltpu.VMEM((2000,), jnp.int32),
            pltpu.VMEM((2000,), jnp.int32),
            pltpu.VMEM((2000,), jnp.int32),
            pltpu.VMEM((80,), jnp.int32),
            pltpu.VMEM((80,), jnp.int32),
            pltpu.VMEM((80,), jnp.int32),
            pltpu.VMEM((80,), jnp.int32),
            pltpu.VMEM((80,), jnp.int32),
            pltpu.VMEM((80,), jnp.float32),
            pltpu.VMEM((80, _H1), jnp.float32),
            pltpu.VMEM((80,), jnp.int32),
            pltpu.VMEM((80,), jnp.int32),
            pltpu.VMEM((80,), jnp.int32),
            pltpu.VMEM((80,), jnp.int32),
            pltpu.VMEM((80,), jnp.int32),
            pltpu.VMEM((80,), jnp.float32),
            pltpu.VMEM((80, _H1), jnp.float32),
            pltpu.SemaphoreType.DMA,
            pltpu.SemaphoreType.DMA,
        ],
    )(_sc_rgcn_body)
    return kfn(srcv, dstv, etv, tbl)


def _sc_score_body(dstv, srcv, qlo, qhi, klo, khi, sco, pmax,
                   dstbA, srcbA, dstbB, srcbB,
                   qdlA, qdhA, kslA, kshA, qdlB, qdhB, kslB, kshB,
                   scv, mbuf, semA, semB):
    c = lax.axis_index("c")
    s = lax.axis_index("s")
    w = c * _NS + s
    e0 = w * 10000
    nch = 125
    neg = jnp.full((16,), -3.0e38, jnp.float32)
    i16 = lax.iota(jnp.int32, 16)

    bufs = ((dstbA, srcbA, qdlA, qdhA, kslA, kshA, semA),
            (dstbB, srcbB, qdlB, qdhB, kslB, kshB, semB))

    def _start(cix, bi):
        dstb, srcb, qdl, qdh, ksl, ksh, sem = bufs[bi]
        b0 = e0 + cix * 80
        pltpu.sync_copy(dstv.at[pl.ds(b0, 80)], dstb)
        pltpu.sync_copy(srcv.at[pl.ds(b0, 80)], srcb)
        pltpu.async_copy(qlo.at[dstb], qdl, sem)
        pltpu.async_copy(qhi.at[dstb], qdh, sem)
        pltpu.async_copy(klo.at[srcb], ksl, sem)
        pltpu.async_copy(khi.at[srcb], ksh, sem)

    def _wait(bi):
        dstb, srcb, qdl, qdh, ksl, ksh, sem = bufs[bi]
        pltpu.make_async_copy(qlo.at[dstb], qdl, sem).wait()
        pltpu.make_async_copy(qhi.at[dstb], qdh, sem).wait()
        pltpu.make_async_copy(klo.at[srcb], ksl, sem).wait()
        pltpu.make_async_copy(khi.at[srcb], ksh, sem).wait()

    def _compute(cix, bi, carry):
        _, _, qdl, qdh, ksl, ksh, _ = bufs[bi]
        b0 = e0 + cix * 80

        def _sub(t, carry2):
            svec = [jnp.zeros((16,), jnp.float32) for _ in range(4)]
            for jj in range(16):
                e = t * 16 + jj
                lane = i16 == jj
                for h, (qref, kref) in enumerate(
                        ((qdl, ksl), (qdl, ksl), (qdh, ksh), (qdh, ksh))):
                    base = (h % 2) * 64
                    p = jnp.zeros((16,), jnp.float32)
                    for v in range(4):
                        cs = pl.ds(base + v * 16, 16)
                        p = p + qref[e, cs] * kref[e, cs]
                    for st in (8, 4, 2, 1):
                        p = p + p[i16 ^ st]
                    svec[h] = jnp.where(lane, p * 0.125, svec[h])
            sl = pl.ds(t * 16, 16)
            out2 = []
            for h in range(4):
                scv[h, sl] = svec[h]
                out2.append(jnp.maximum(carry2[h], svec[h]))
            return tuple(out2)
        carry = lax.fori_loop(0, 5, _sub, carry)
        for h in range(4):
            pltpu.sync_copy(scv.at[h], sco.at[pl.ds(h * _E + b0, 80)])
        return carry

    _start(0, 0)

    def _pair(i, carry):
        _wait(0)
        _start(2 * i + 1, 1)
        carry = _compute(2 * i, 0, carry)
        _wait(1)
        _start(2 * i + 2, 0)
        carry = _compute(2 * i + 1, 1, carry)
        return carry

    carry = lax.fori_loop(0, (nch - 1) // 2, _pair, (neg, neg, neg, neg))
    _wait(0)
    m0, m1, m2, m3 = _compute(nch - 1, 0, carry)
    mbuf[...] = jnp.maximum(jnp.maximum(m0, m1), jnp.maximum(m2, m3))
    pltpu.sync_copy(mbuf, pmax.at[pl.ds(w * 16, 16)])


def _sc_score(dstv, srcv, qlo, qhi, klo, khi):
    mesh = plsc.VectorSubcoreMesh(core_axis_name="c", subcore_axis_name="s")
    row = lambda: pltpu.VMEM((80, 128), jnp.float32)
    idx = lambda: pltpu.VMEM((80,), jnp.int32)
    kfn = functools.partial(
        pl.kernel,
        out_type=(jax.ShapeDtypeStruct((4 * _E,), jnp.float32),
                  jax.ShapeDtypeStruct((_NC * _NS * 16,), jnp.float32)),
        mesh=mesh,
        scratch_types=[
            idx(), idx(), idx(), idx(),
            row(), row(), row(), row(), row(), row(), row(), row(),
            pltpu.VMEM((4, 80), jnp.float32),
            pltpu.VMEM((16,), jnp.float32),
            pltpu.SemaphoreType.DMA,
            pltpu.SemaphoreType.DMA,
        ],
    )(_sc_score_body)
    return kfn(dstv, srcv, qlo, qhi, klo, khi)


def _sc_attn_body(dstv, srcv, v2, sco, pmax, nump, denp,
                  acc_sp, den_sp, zb2, zbd,
                  dstbA, srcbA, idxvA, idxdaA, idxdbA, saA, sbA, vbA,
                  dstbB, srcbB, idxvB, idxdaB, idxdbB, saB, sbB, vbB,
                  wab, wbb, rowsbA, rowsbB, mxv, semA, semB, semSA, semSB):
    c = lax.axis_index("c")
    s = lax.axis_index("s")

    pltpu.sync_copy(pmax, mxv)
    m = mxv[pl.ds(0, 16)]
    for r in range(1, 32):
        m = jnp.maximum(m, mxv[pl.ds(r * 16, 16)])
    i16g = lax.iota(jnp.int32, 16)
    for st in (8, 4, 2, 1):
        m = jnp.maximum(m, m[i16g ^ st])
    gmax = m

    _fill(zb2, 64, 0.0)
    _fill(zbd, 157, 0.0)
    _zero_rows(zb2, acc_sp, s)
    pltpu.sync_copy(zbd, den_sp.at[pl.ds(s * 2512, 2512)])
    plsc.subcore_barrier()

    bufs = ((dstbA, srcbA, idxvA, idxdaA, idxdbA, saA, sbA, vbA, semA),
            (dstbB, srcbB, idxvB, idxdaB, idxdbB, saB, sbB, vbB, semB))
    sbufs = ((rowsbA, semSA), (rowsbB, semSB))

    def _start(cix, bi, drain=False):
        dstb, srcb, idxv, idxda, idxdb, sa, sb, vb, sem = bufs[bi]
        if drain:
            rowsb, semS = sbufs[bi]
            pltpu.make_async_copy(rowsb, acc_sp.at[dstb], semS).wait()
        b0 = s * 20000 + cix * 80
        pltpu.sync_copy(dstv.at[pl.ds(b0, 80)], dstb)
        pltpu.sync_copy(srcv.at[pl.ds(b0, 80)], srcb)

        def _ix(i, _):
            sl = pl.ds(i * 16, 16)
            idxv[sl] = srcb[sl] + c * _N
            idxda[sl] = dstb[sl] * 4 + 2 * c
            idxdb[sl] = dstb[sl] * 4 + (2 * c + 1)
            return 0
        lax.fori_loop(0, 5, _ix, 0)
        pltpu.async_copy(v2.at[idxv], vb, sem)
        pltpu.sync_copy(
            sco.at[pl.ds(pl.multiple_of(2 * c * _E + b0, 8), 80)], sa)
        pltpu.sync_copy(
            sco.at[pl.ds(pl.multiple_of((2 * c + 1) * _E + b0, 8), 80)], sb)

    def _wait(bi):
        _, _, idxv, _, _, _, _, vb, sem = bufs[bi]
        pltpu.make_async_copy(v2.at[idxv], vb, sem).wait()

    def _compute(bi):
        dstb, srcb, idxv, idxda, idxdb, sa, sb, vb, sem = bufs[bi]
        rowsb, semS = sbufs[bi]

        def _rows(i, _):
            sl = pl.ds(i * 16, 16)
            wa16 = jnp.exp(sa[sl] - gmax)
            wb16 = jnp.exp(sb[sl] - gmax)
            wab[sl] = wa16
            wbb[sl] = wb16
            for jj in range(16):
                e = i * 16 + jj
                wav = _splat(wa16, jj)
                wbv = _splat(wb16, jj)
                for vv in range(4):
                    cs = pl.ds(vv * 16, 16)
                    rowsb[e, cs] = vb[e, cs] * wav
                for vv in range(4, 8):
                    cs = pl.ds(vv * 16, 16)
                    rowsb[e, cs] = vb[e, cs] * wbv
            return 0
        lax.fori_loop(0, 5, _rows, 0)

        pltpu.async_copy(rowsb, acc_sp.at[dstb], semS, add=True)
        pltpu.sync_copy(wab, den_sp.at[idxda], add=True)
        pltpu.sync_copy(wbb, den_sp.at[idxdb], add=True)

    _start(0, 0)
    _wait(0)
    _start(1, 1)
    _compute(0)
    _wait(1)
    _start(2, 0, drain=True)
    _compute(1)

    def _pair(i, _):
        _wait(0)
        _start(2 * i + 3, 1, drain=True)
        _compute(0)
        _wait(1)
        _start(2 * i + 4, 0, drain=True)
        _compute(1)
        return 0
    lax.fori_loop(0, 123, _pair, 0)
    _wait(0)
    _start(249, 1, drain=True)
    _compute(0)
    _wait(1)
    _compute(1)
    pltpu.make_async_copy(rowsbA, acc_sp.at[dstbA], semSA).wait()
    pltpu.make_async_copy(rowsbB, acc_sp.at[dstbB], semSB).wait()
    plsc.subcore_barrier()

    _tile_rows(lambda r0, nr: pltpu.sync_copy(
        acc_sp.at[pl.ds(r0, nr)], nump.at[c, pl.ds(r0, nr)]), s)
    pltpu.sync_copy(den_sp.at[pl.ds(s * 2512, 2512)], zbd)
    pltpu.sync_copy(zbd, denp.at[pl.ds(c * _D4 + s * 2512, 2512)])


def _sc_attn(dstv, srcv, v2, sco, pmax):
    mesh = plsc.VectorSubcoreMesh(core_axis_name="c", subcore_axis_name="s")
    idx = lambda: pltpu.VMEM((80,), jnp.int32)
    f80 = lambda: pltpu.VMEM((80,), jnp.float32)
    row = lambda: pltpu.VMEM((80, _H1), jnp.float32)
    kfn = functools.partial(
        pl.kernel,
        out_type=(jax.ShapeDtypeStruct((_NC, _N, _H1), jnp.float32),
                  jax.ShapeDtypeStruct((_NC * _D4,), jnp.float32)),
        mesh=mesh,
        scratch_types=[
            pltpu.VMEM_SHARED((_N, _H1), jnp.float32),
            pltpu.VMEM_SHARED((_D4,), jnp.float32),
            pltpu.VMEM((8, _H1), jnp.float32),
            pltpu.VMEM((2512,), jnp.float32),
            idx(), idx(), idx(), idx(), idx(), f80(), f80(), row(),
            idx(), idx(), idx(), idx(), idx(), f80(), f80(), row(),
            f80(),
            f80(),
            row(),
            row(),
            pltpu.VMEM((512,), jnp.float32),
            pltpu.SemaphoreType.DMA,
            pltpu.SemaphoreType.DMA,
            pltpu.SemaphoreType.DMA,
            pltpu.SemaphoreType.DMA,
        ],
    )(_sc_attn_body)
    return kfn(dstv, srcv, v2, sco, pmax)


def _tc_head_body(n0_ref, n1_ref, d0_ref, d1_ref, sk_ref, bsk_ref,
                  h_ref, st_ref):
    i = pl.program_id(0)
    den = d0_ref[...] + d1_ref[...] + 1e-16
    parts = []
    for h in range(4):
        nref = n0_ref if h < 2 else n1_ref
        col = (h % 2) * 64
        parts.append(nref[:, col:col + 64] / den[:, h:h + 1])
    h_val = jnp.concatenate(parts, axis=1) + sk_ref[...] + bsk_ref[...]
    h_ref[...] = h_val

    @pl.when(i == 0)
    def _():
        st_ref[...] = jnp.zeros_like(st_ref)
    st_ref[0:1, :] += jnp.sum(h_val, axis=0, keepdims=True)
    st_ref[1:2, :] += jnp.sum(h_val * h_val, axis=0, keepdims=True)


def _tc_head(n0, n1, d0, d1, sk, bsk, bn=2000):
    n = n0.shape[0]
    return pl.pallas_call(
        _tc_head_body,
        grid=(n // bn,),
        in_specs=[pl.BlockSpec((bn, 128), lambda i: (i, 0)),
                  pl.BlockSpec((bn, 128), lambda i: (i, 0)),
                  pl.BlockSpec((bn, 4), lambda i: (i, 0)),
                  pl.BlockSpec((bn, 4), lambda i: (i, 0)),
                  pl.BlockSpec((bn, 256), lambda i: (i, 0)),
                  pl.BlockSpec((1, 256), lambda i: (0, 0))],
        out_specs=(pl.BlockSpec((bn, 256), lambda i: (i, 0)),
                   pl.BlockSpec((8, 256), lambda i: (0, 0))),
        out_shape=(jax.ShapeDtypeStruct((n, 256), jnp.float32),
                   jax.ShapeDtypeStruct((8, 256), jnp.float32)),
    )(n0, n1, d0, d1, sk, bsk)


def _tc_bn_body(h_ref, st_ref, g_ref, b_ref, o_ref):
    h = h_ref[...]
    n = jnp.float32(_N)
    mean = st_ref[0:1, :] / n
    var = st_ref[1:2, :] / n - mean * mean
    y = (h - mean) / jnp.sqrt(var + 1e-5) * g_ref[...] + b_ref[...]
    o_ref[...] = jnp.where(y > 0, y, 0.01 * y)


def _tc_bn(h, st, g, b, bn=2000):
    n = h.shape[0]
    return pl.pallas_call(
        _tc_bn_body,
        grid=(n // bn,),
        in_specs=[pl.BlockSpec((bn, 256), lambda i: (i, 0)),
                  pl.BlockSpec((8, 256), lambda i: (0, 0)),
                  pl.BlockSpec((1, 256), lambda i: (0, 0)),
                  pl.BlockSpec((1, 256), lambda i: (0, 0))],
        out_specs=pl.BlockSpec((bn, 256), lambda i: (i, 0)),
        out_shape=jax.ShapeDtypeStruct((n, 256), jnp.float32),
    )(h, st, g, b)


def kernel(node_features, node_type, edge_index, edge_type, W_rel, W_root,
           b_rgcn, Wq, bq, Wk, bk, Wv, bv, Wskip, bskip, gamma, beta):
    del node_type
    srcv = edge_index[0].astype(jnp.int32)
    dstv = edge_index[1].astype(jnp.int32)
    etv = edge_type.astype(jnp.int32)

    w2 = W_rel.transpose(1, 0, 2).reshape(_G, _R * _H1)
    wcat1 = jnp.concatenate([w2, W_root], axis=1)
    mm1 = _tc_matmul(node_features, wcat1)
    tbl = mm1[:, :_R * _H1].reshape(_N * _R, _H1)
    xr = mm1[:, _R * _H1:]

    aggp = _sc_rgcn(srcv, dstv, etv, tbl)

    wcat2 = jnp.concatenate([Wq, Wk, Wv, Wskip], axis=1)
    bcat2 = jnp.concatenate([bq, bk, bv, bskip]).reshape(1, 1024)
    qkvs = _tc_fuse(aggp[0], aggp[1], xr, b_rgcn.reshape(1, _H1),
                    wcat2, bcat2)
    qlo, qhi = qkvs[:, 0:128], qkvs[:, 128:256]
    klo, khi = qkvs[:, 256:384], qkvs[:, 384:512]
    v2 = jnp.concatenate([qkvs[:, 512:640], qkvs[:, 640:768]], axis=0)
    sk = qkvs[:, 768:1024]

    sco, pmax = _sc_score(dstv, srcv, qlo, qhi, klo, khi)

    nump, denp = _sc_attn(dstv, srcv, v2, sco, pmax)

    d0 = denp[:_N * 4].reshape(_N, 4)
    d1 = denp[_D4:_D4 + _N * 4].reshape(_N, 4)
    h, st = _tc_head(nump[0], nump[1], d0, d1, sk, bskip.reshape(1, 256))
    out = _tc_bn(h, st, gamma.reshape(1, 256), beta.reshape(1, 256))
    return out


# speedup vs baseline: 1.1715x; 1.0006x over previous
import functools
import numpy as np
import jax
import jax.numpy as jnp
from jax import lax
from jax.experimental import pallas as pl
from jax.experimental.pallas import tpu as pltpu, tpu_sc as plsc

_N = 10000
_E = 320000
_G = 128
_H1 = 128
_H2 = 64
_R = 8
_HEADS = 4
_NC = 2
_NS = 16

_D4 = 40192


def _splat(vec16, j):
    return jnp.broadcast_to(vec16[j], (16,))


def _fill(ref, nvec, value):
    v = jnp.full((16,), value, jnp.float32)
    if len(ref.shape) == 1:
        def _b(i, _):
            ref[pl.ds(i * 16, 16)] = v
            return 0
    else:
        ncol = ref.shape[1] // 16

        def _b(i, _):
            ref[i // ncol, pl.ds((i % ncol) * 16, 16)] = v
            return 0
    lax.fori_loop(0, nvec, _b, 0)


def _zero_rows(zb8, sp_ref, s):
    def _b(j, _):
        pltpu.sync_copy(zb8, sp_ref.at[pl.ds(s * 624 + j * 8, 8)])
        return 0
    lax.fori_loop(0, 78, _b, 0)

    @pl.when(s == _NS - 1)
    def _():
        pltpu.sync_copy(zb8, sp_ref.at[pl.ds(9984, 8)])
        pltpu.sync_copy(zb8, sp_ref.at[pl.ds(9992, 8)])


def _tile_rows(sync_fn, s):
    def _b(j, _):
        sync_fn(s * 624 + j * 208, 208)
        return 0
    lax.fori_loop(0, 3, _b, 0)

    @pl.when(s == _NS - 1)
    def _():
        sync_fn(9984, 16)


def _tc_matmul_body(x_ref, w_ref, o_ref):
    o_ref[...] = jnp.dot(x_ref[...], w_ref[...],
                         preferred_element_type=jnp.float32)


def _tc_matmul(x, w, bn=2000):
    n, kdim = x.shape
    m = w.shape[1]
    return pl.pallas_call(
        _tc_matmul_body,
        grid=(n // bn,),
        in_specs=[pl.BlockSpec((bn, kdim), lambda i: (i, 0)),
                  pl.BlockSpec((kdim, m), lambda i: (0, 0))],
        out_specs=pl.BlockSpec((bn, m), lambda i: (i, 0)),
        out_shape=jax.ShapeDtypeStruct((n, m), jnp.float32),
    )(x, w)


def _tc_fuse_body(a0_ref, a1_ref, xr_ref, b_ref, w_ref, b2_ref, o_ref):
    x2 = a0_ref[...] + a1_ref[...] + xr_ref[...] + b_ref[...]
    o_ref[...] = jnp.dot(x2, w_ref[...],
                         preferred_element_type=jnp.float32) + b2_ref[...]


def _tc_fuse(a0, a1, xr, b, w, b2, bn=2000):
    n, kdim = a0.shape
    m = w.shape[1]
    return pl.pallas_call(
        _tc_fuse_body,
        grid=(n // bn,),
        in_specs=[pl.BlockSpec((bn, kdim), lambda i: (i, 0)),
                  pl.BlockSpec((bn, kdim), lambda i: (i, 0)),
                  pl.BlockSpec((bn, kdim), lambda i: (i, 0)),
                  pl.BlockSpec((1, kdim), lambda i: (0, 0)),
                  pl.BlockSpec((kdim, m), lambda i: (0, 0)),
                  pl.BlockSpec((1, m), lambda i: (0, 0))],
        out_specs=pl.BlockSpec((bn, m), lambda i: (i, 0)),
        out_shape=jax.ShapeDtypeStruct((n, m), jnp.float32),
    )(a0, a1, xr, b, w, b2)


def _sc_rgcn_body(srcv, dstv, etv, tbl, aggp,
                  cnt_sp, agg_sp, onesb, zb2, zbd,
                  dstb, etb, idxc,
                  srcbA, etb2A, dstb2A, idxmA, idxc2A, cntgA, msgA,
                  srcbB, etb2B, dstb2B, idxmB, idxc2B, cntgB, msgB,
                  semA, semB):
    c = lax.axis_index("c")
    s = lax.axis_index("s")

    _fill(onesb, 125, 1.0)
    _fill(zb2, 64, 0.0)
    _fill(zbd, 157, 0.0)

    pltpu.sync_copy(zbd, cnt_sp.at[pl.ds(s * 5008, 2512)])
    pltpu.sync_copy(zbd, cnt_sp.at[pl.ds(s * 5008 + 2496, 2512)])
    _zero_rows(zb2, agg_sp, s)
    plsc.subcore_barrier()

    def _cnt(j, _):
        b0 = s * 20000 + j * 2000
        pltpu.sync_copy(dstv.at[pl.ds(b0, 2000)], dstb)
        pltpu.sync_copy(etv.at[pl.ds(b0, 2000)], etb)

        def _ix(i, _):
            sl = pl.ds(i * 16, 16)
            idxc[sl] = dstb[sl] * _R + etb[sl]
            return 0
        lax.fori_loop(0, 125, _ix, 0)
        pltpu.sync_copy(onesb, cnt_sp.at[idxc], add=True)
        return 0
    lax.fori_loop(0, 10, _cnt, 0)
    plsc.subcore_barrier()

    bufs = ((srcbA, etb2A, dstb2A, idxmA, idxc2A, cntgA, msgA, semA),
            (srcbB, etb2B, dstb2B, idxmB, idxc2B, cntgB, msgB, semB))

    def _start(cix, bi):
        srcb, etb2, dstb2, idxm, idxc2, cntg, msg, sem = bufs[bi]
        b0 = c * 160000 + s * 10000 + cix * 80
        pltpu.sync_copy(srcv.at[pl.ds(b0, 80)], srcb)
        pltpu.sync_copy(etv.at[pl.ds(b0, 80)], etb2)
        pltpu.sync_copy(dstv.at[pl.ds(b0, 80)], dstb2)

        def _ix2(i, _):
            sl = pl.ds(i * 16, 16)
            idxm[sl] = srcb[sl] * _R + etb2[sl]
            idxc2[sl] = dstb2[sl] * _R + etb2[sl]
            return 0
        lax.fori_loop(0, 5, _ix2, 0)
        pltpu.async_copy(tbl.at[idxm], msg, sem)

    def _wait(bi):
        srcb, etb2, dstb2, idxm, idxc2, cntg, msg, sem = bufs[bi]
        pltpu.make_async_copy(tbl.at[idxm], msg, sem).wait()
        pltpu.async_copy(cnt_sp.at[idxc2], cntg, sem).wait()

    def _compute(bi):
        srcb, etb2, dstb2, idxm, idxc2, cntg, msg, sem = bufs[bi]

        def _scale(i, _):
            c16 = cntg[pl.ds(i * 16, 16)]
            n16 = 1.0 / jnp.maximum(c16, 1.0)
            for jj in range(16):
                e = i * 16 + jj
                nv = _splat(n16, jj)
                for vv in range(8):
                    sl = pl.ds(vv * 16, 16)
                    msg[e, sl] = msg[e, sl] * nv
            return 0
        lax.fori_loop(0, 5, _scale, 0)
        pltpu.sync_copy(msg, agg_sp.at[dstb2], add=True)

    _start(0, 0)

    def _pair(i, _):
        _wait(0)
        _start(2 * i + 1, 1)
        _compute(0)
        _wait(1)
        _start(2 * i + 2, 0)
        _compute(1)
        return 0
    lax.fori_loop(0, 62, _pair, 0)
    _wait(0)
    _compute(0)
    plsc.subcore_barrier()

    _tile_rows(lambda r0, nr: pltpu.sync_copy(
        agg_sp.at[pl.ds(r0, nr)], aggp.at[c, pl.ds(r0, nr)]), s)


def _sc_rgcn(srcv, dstv, etv, tbl):
    mesh = plsc.VectorSubcoreMesh(core_axis_name="c", subcore_axis_name="s")
    kfn = functools.partial(
        pl.kernel,
        out_type=jax.ShapeDtypeStruct((_NC, _N, _H1), jnp.float32),
        mesh=mesh,
        scratch_types=[
            pltpu.VMEM_SHARED((80128,), jnp.float32),
            pltpu.VMEM_SHARED((_N, _H1), jnp.float32),
            pltpu.VMEM((2000,), jnp.float32),
            pltpu.VMEM((8, _H1), jnp.float32),
            pltpu.VMEM((2512,), jnp.float32),
            pltpu.VMEM((2000,), jnp.int32),
            pltpu.VMEM((2000,), jnp.int32),
            pltpu.VMEM((2000,), jnp.int32),
            pltpu.VMEM((80,), jnp.int32),
            pltpu.VMEM((80,), jnp.int32),
            pltpu.VMEM((80,), jnp.int32),
            pltpu.VMEM((80,), jnp.int32),
            pltpu.VMEM((80,), jnp.int32),
            pltpu.VMEM((80,), jnp.float32),
            pltpu.VMEM((80, _H1), jnp.float32),
            pltpu.VMEM((80,), jnp.int32),
            pltpu.VMEM((80,), jnp.int32),
            pltpu.VMEM((80,), jnp.int32),
            pltpu.VMEM((80,), jnp.int32),
            pltpu.VMEM((80,), jnp.int32),
            pltpu.VMEM((80,), jnp.float32),
            pltpu.VMEM((80, _H1), jnp.float32),
            pltpu.SemaphoreType.DMA,
            pltpu.SemaphoreType.DMA,
        ],
    )(_sc_rgcn_body)
    return kfn(srcv, dstv, etv, tbl)


def _sc_score_body(dstv, srcv, qlo, qhi, klo, khi, sco, pmax,
                   dstbA, srcbA, dstbB, srcbB,
                   qdlA, qdhA, kslA, kshA, qdlB, qdhB, kslB, kshB,
                   scv, mbuf, semA, semB):
    c = lax.axis_index("c")
    s = lax.axis_index("s")
    w = c * _NS + s
    e0 = w * 10000
    nch = 125
    neg = jnp.full((16,), -3.0e38, jnp.float32)
    i16 = lax.iota(jnp.int32, 16)

    bufs = ((dstbA, srcbA, qdlA, qdhA, kslA, kshA, semA),
            (dstbB, srcbB, qdlB, qdhB, kslB, kshB, semB))

    def _start(cix, bi):
        dstb, srcb, qdl, qdh, ksl, ksh, sem = bufs[bi]
        b0 = e0 + cix * 80
        pltpu.sync_copy(dstv.at[pl.ds(b0, 80)], dstb)
        pltpu.sync_copy(srcv.at[pl.ds(b0, 80)], srcb)
        pltpu.async_copy(qlo.at[dstb], qdl, sem)
        pltpu.async_copy(qhi.at[dstb], qdh, sem)
        pltpu.async_copy(klo.at[srcb], ksl, sem)
        pltpu.async_copy(khi.at[srcb], ksh, sem)

    def _wait(bi):
        dstb, srcb, qdl, qdh, ksl, ksh, sem = bufs[bi]
        pltpu.make_async_copy(qlo.at[dstb], qdl, sem).wait()
        pltpu.make_async_copy(qhi.at[dstb], qdh, sem).wait()
        pltpu.make_async_copy(klo.at[srcb], ksl, sem).wait()
        pltpu.make_async_copy(khi.at[srcb], ksh, sem).wait()

    def _compute(cix, bi, carry):
        _, _, qdl, qdh, ksl, ksh, _ = bufs[bi]
        b0 = e0 + cix * 80

        def _sub(t, carry2):
            svec = [jnp.zeros((16,), jnp.float32) for _ in range(4)]
            for jj in range(16):
                e = t * 16 + jj
                lane = i16 == jj
                for h, (qref, kref) in enumerate(
                        ((qdl, ksl), (qdl, ksl), (qdh, ksh), (qdh, ksh))):
                    base = (h % 2) * 64
                    p = jnp.zeros((16,), jnp.float32)
                    for v in range(4):
                        cs = pl.ds(base + v * 16, 16)
                        p = p + qref[e, cs] * kref[e, cs]
                    for st in (8, 4, 2, 1):
                        p = p + p[i16 ^ st]
                    svec[h] = jnp.where(lane, p * 0.125, svec[h])
            sl = pl.ds(t * 16, 16)
            out2 = []
            for h in range(4):
                scv[h, sl] = svec[h]
                out2.append(jnp.maximum(carry2[h], svec[h]))
            return tuple(out2)
        carry = lax.fori_loop(0, 5, _sub, carry)
        for h in range(4):
            pltpu.sync_copy(scv.at[h], sco.at[pl.ds(h * _E + b0, 80)])
        return carry

    _start(0, 0)

    def _pair(i, carry):
        _wait(0)
        _start(2 * i + 1, 1)
        carry = _compute(2 * i, 0, carry)
        _wait(1)
        _start(2 * i + 2, 0)
        carry = _compute(2 * i + 1, 1, carry)
        return carry

    carry = lax.fori_loop(0, (nch - 1) // 2, _pair, (neg, neg, neg, neg))
    _wait(0)
    m0, m1, m2, m3 = _compute(nch - 1, 0, carry)
    mbuf[...] = jnp.maximum(jnp.maximum(m0, m1), jnp.maximum(m2, m3))
    pltpu.sync_copy(mbuf, pmax.at[pl.ds(w * 16, 16)])


def _sc_score(dstv, srcv, qlo, qhi, klo, khi):
    mesh = plsc.VectorSubcoreMesh(core_axis_name="c", subcore_axis_name="s")
    row = lambda: pltpu.VMEM((80, 128), jnp.float32)
    idx = lambda: pltpu.VMEM((80,), jnp.int32)
    kfn = functools.partial(
        pl.kernel,
        out_type=(jax.ShapeDtypeStruct((4 * _E,), jnp.float32),
                  jax.ShapeDtypeStruct((_NC * _NS * 16,), jnp.float32)),
        mesh=mesh,
        scratch_types=[
            idx(), idx(), idx(), idx(),
            row(), row(), row(), row(), row(), row(), row(), row(),
            pltpu.VMEM((4, 80), jnp.float32),
            pltpu.VMEM((16,), jnp.float32),
            pltpu.SemaphoreType.DMA,
            pltpu.SemaphoreType.DMA,
        ],
    )(_sc_score_body)
    return kfn(dstv, srcv, qlo, qhi, klo, khi)


def _sc_attn_body(dstv, srcv, v2, sco, pmax, nump, denp,
                  acc_sp, den_sp, zb2, zbd,
                  dstbA, srcbA, idxvA, idxdA, saA, sbA, vbA,
                  dstbB, srcbB, idxvB, idxdB, saB, sbB, vbB,
                  wdA, wdB, rowsbA, rowsbB, mxv, semA, semB, semSA, semSB):
    c = lax.axis_index("c")
    s = lax.axis_index("s")

    pltpu.sync_copy(pmax, mxv)
    m = mxv[pl.ds(0, 16)]
    for r in range(1, 32):
        m = jnp.maximum(m, mxv[pl.ds(r * 16, 16)])
    i16g = lax.iota(jnp.int32, 16)
    for st in (8, 4, 2, 1):
        m = jnp.maximum(m, m[i16g ^ st])
    gmax = m

    _fill(zb2, 64, 0.0)
    _fill(zbd, 157, 0.0)
    _zero_rows(zb2, acc_sp, s)
    pltpu.sync_copy(zbd, den_sp.at[pl.ds(s * 2512, 2512)])
    plsc.subcore_barrier()

    bufs = ((dstbA, srcbA, idxvA, idxdA, saA, sbA, vbA, wdA, semA),
            (dstbB, srcbB, idxvB, idxdB, saB, sbB, vbB, wdB, semB))
    sbufs = ((rowsbA, semSA), (rowsbB, semSB))

    def _start(cix, bi, drain=False):
        dstb, srcb, idxv, idxd, sa, sb, vb, wd, sem = bufs[bi]
        if drain:
            rowsb, semS = sbufs[bi]
            pltpu.make_async_copy(rowsb, acc_sp.at[dstb], semS).wait()
        b0 = s * 20000 + cix * 80
        pltpu.sync_copy(dstv.at[pl.ds(b0, 80)], dstb)
        pltpu.sync_copy(srcv.at[pl.ds(b0, 80)], srcb)

        def _ix(i, _):
            sl = pl.ds(i * 16, 16)
            sl2 = pl.ds(80 + i * 16, 16)
            idxv[sl] = srcb[sl] + c * _N
            idxd[sl] = dstb[sl] * 4 + 2 * c
            idxd[sl2] = dstb[sl] * 4 + (2 * c + 1)
            return 0
        lax.fori_loop(0, 5, _ix, 0)
        pltpu.async_copy(v2.at[idxv], vb, sem)
        pltpu.sync_copy(
            sco.at[pl.ds(pl.multiple_of(2 * c * _E + b0, 8), 80)], sa)
        pltpu.sync_copy(
            sco.at[pl.ds(pl.multiple_of((2 * c + 1) * _E + b0, 8), 80)], sb)

    def _wait(bi):
        _, _, idxv, _, _, _, vb, _, sem = bufs[bi]
        pltpu.make_async_copy(v2.at[idxv], vb, sem).wait()

    def _compute(bi):
        dstb, srcb, idxv, idxd, sa, sb, vb, wd, sem = bufs[bi]
        rowsb, semS = sbufs[bi]

        def _rows(i, _):
            sl = pl.ds(i * 16, 16)
            wa16 = jnp.exp(sa[sl] - gmax)
            wb16 = jnp.exp(sb[sl] - gmax)
            wd[sl] = wa16
            wd[pl.ds(80 + i * 16, 16)] = wb16
            for jj in range(16):
                e = i * 16 + jj
                wav = _splat(wa16, jj)
                wbv = _splat(wb16, jj)
                for vv in range(4):
                    cs = pl.ds(vv * 16, 16)
                    rowsb[e, cs] = vb[e, cs] * wav
                for vv in range(4, 8):
                    cs = pl.ds(vv * 16, 16)
                    rowsb[e, cs] = vb[e, cs] * wbv
            return 0
        lax.fori_loop(0, 5, _rows, 0)

        pltpu.async_copy(rowsb, acc_sp.at[dstb], semS, add=True)
        pltpu.sync_copy(wd, den_sp.at[idxd], add=True)

    _start(0, 0)
    _wait(0)
    _start(1, 1)
    _compute(0)
    _wait(1)
    _start(2, 0, drain=True)
    _compute(1)

    def _pair(i, _):
        _wait(0)
        _start(2 * i + 3, 1, drain=True)
        _compute(0)
        _wait(1)
        _start(2 * i + 4, 0, drain=True)
        _compute(1)
        return 0
    lax.fori_loop(0, 123, _pair, 0)
    _wait(0)
    _start(249, 1, drain=True)
    _compute(0)
    _wait(1)
    _compute(1)
    pltpu.make_async_copy(rowsbA, acc_sp.at[dstbA], semSA).wait()
    pltpu.make_async_copy(rowsbB, acc_sp.at[dstbB], semSB).wait()
    plsc.subcore_barrier()

    _tile_rows(lambda r0, nr: pltpu.sync_copy(
        acc_sp.at[pl.ds(r0, nr)], nump.at[c, pl.ds(r0, nr)]), s)
    pltpu.sync_copy(den_sp.at[pl.ds(s * 2512, 2512)], zbd)
    pltpu.sync_copy(zbd, denp.at[pl.ds(c * _D4 + s * 2512, 2512)])


def _sc_attn(dstv, srcv, v2, sco, pmax):
    mesh = plsc.VectorSubcoreMesh(core_axis_name="c", subcore_axis_name="s")
    idx = lambda: pltpu.VMEM((80,), jnp.int32)
    f80 = lambda: pltpu.VMEM((80,), jnp.float32)
    row = lambda: pltpu.VMEM((80, _H1), jnp.float32)
    kfn = functools.partial(
        pl.kernel,
        out_type=(jax.ShapeDtypeStruct((_NC, _N, _H1), jnp.float32),
                  jax.ShapeDtypeStruct((_NC * _D4,), jnp.float32)),
        mesh=mesh,
        scratch_types=[
            pltpu.VMEM_SHARED((_N, _H1), jnp.float32),
            pltpu.VMEM_SHARED((_D4,), jnp.float32),
            pltpu.VMEM((8, _H1), jnp.float32),
            pltpu.VMEM((2512,), jnp.float32),
            idx(), idx(), idx(), pltpu.VMEM((160,), jnp.int32),
            f80(), f80(), row(),
            idx(), idx(), idx(), pltpu.VMEM((160,), jnp.int32),
            f80(), f80(), row(),
            pltpu.VMEM((160,), jnp.float32),
            pltpu.VMEM((160,), jnp.float32),
            row(),
            row(),
            pltpu.VMEM((512,), jnp.float32),
            pltpu.SemaphoreType.DMA,
            pltpu.SemaphoreType.DMA,
            pltpu.SemaphoreType.DMA,
            pltpu.SemaphoreType.DMA,
        ],
    )(_sc_attn_body)
    return kfn(dstv, srcv, v2, sco, pmax)


def _tc_head_body(n0_ref, n1_ref, d0_ref, d1_ref, sk_ref, bsk_ref,
                  h_ref, st_ref):
    i = pl.program_id(0)
    den = d0_ref[...] + d1_ref[...] + 1e-16
    parts = []
    for h in range(4):
        nref = n0_ref if h < 2 else n1_ref
        col = (h % 2) * 64
        parts.append(nref[:, col:col + 64] / den[:, h:h + 1])
    h_val = jnp.concatenate(parts, axis=1) + sk_ref[...] + bsk_ref[...]
    h_ref[...] = h_val

    @pl.when(i == 0)
    def _():
        st_ref[...] = jnp.zeros_like(st_ref)
    st_ref[0:1, :] += jnp.sum(h_val, axis=0, keepdims=True)
    st_ref[1:2, :] += jnp.sum(h_val * h_val, axis=0, keepdims=True)


def _tc_head(n0, n1, d0, d1, sk, bsk, bn=2000):
    n = n0.shape[0]
    return pl.pallas_call(
        _tc_head_body,
        grid=(n // bn,),
        in_specs=[pl.BlockSpec((bn, 128), lambda i: (i, 0)),
                  pl.BlockSpec((bn, 128), lambda i: (i, 0)),
                  pl.BlockSpec((bn, 4), lambda i: (i, 0)),
                  pl.BlockSpec((bn, 4), lambda i: (i, 0)),
                  pl.BlockSpec((bn, 256), lambda i: (i, 0)),
                  pl.BlockSpec((1, 256), lambda i: (0, 0))],
        out_specs=(pl.BlockSpec((bn, 256), lambda i: (i, 0)),
                   pl.BlockSpec((8, 256), lambda i: (0, 0))),
        out_shape=(jax.ShapeDtypeStruct((n, 256), jnp.float32),
                   jax.ShapeDtypeStruct((8, 256), jnp.float32)),
    )(n0, n1, d0, d1, sk, bsk)


def _tc_bn_body(h_ref, st_ref, g_ref, b_ref, o_ref):
    h = h_ref[...]
    n = jnp.float32(_N)
    mean = st_ref[0:1, :] / n
    var = st_ref[1:2, :] / n - mean * mean
    y = (h - mean) / jnp.sqrt(var + 1e-5) * g_ref[...] + b_ref[...]
    o_ref[...] = jnp.where(y > 0, y, 0.01 * y)


def _tc_bn(h, st, g, b, bn=2000):
    n = h.shape[0]
    return pl.pallas_call(
        _tc_bn_body,
        grid=(n // bn,),
        in_specs=[pl.BlockSpec((bn, 256), lambda i: (i, 0)),
                  pl.BlockSpec((8, 256), lambda i: (0, 0)),
                  pl.BlockSpec((1, 256), lambda i: (0, 0)),
                  pl.BlockSpec((1, 256), lambda i: (0, 0))],
        out_specs=pl.BlockSpec((bn, 256), lambda i: (i, 0)),
        out_shape=jax.ShapeDtypeStruct((n, 256), jnp.float32),
    )(h, st, g, b)


def kernel(node_features, node_type, edge_index, edge_type, W_rel, W_root,
           b_rgcn, Wq, bq, Wk, bk, Wv, bv, Wskip, bskip, gamma, beta):
    del node_type
    srcv = edge_index[0].astype(jnp.int32)
    dstv = edge_index[1].astype(jnp.int32)
    etv = edge_type.astype(jnp.int32)

    w2 = W_rel.transpose(1, 0, 2).reshape(_G, _R * _H1)
    wcat1 = jnp.concatenate([w2, W_root], axis=1)
    mm1 = _tc_matmul(node_features, wcat1)
    tbl = mm1[:, :_R * _H1].reshape(_N * _R, _H1)
    xr = mm1[:, _R * _H1:]

    aggp = _sc_rgcn(srcv, dstv, etv, tbl)

    wcat2 = jnp.concatenate([Wq, Wk, Wv, Wskip], axis=1)
    bcat2 = jnp.concatenate([bq, bk, bv, bskip]).reshape(1, 1024)
    qkvs = _tc_fuse(aggp[0], aggp[1], xr, b_rgcn.reshape(1, _H1),
                    wcat2, bcat2)
    qlo, qhi = qkvs[:, 0:128], qkvs[:, 128:256]
    klo, khi = qkvs[:, 256:384], qkvs[:, 384:512]
    v2 = jnp.concatenate([qkvs[:, 512:640], qkvs[:, 640:768]], axis=0)
    sk = qkvs[:, 768:1024]

    sco, pmax = _sc_score(dstv, srcv, qlo, qhi, klo, khi)

    nump, denp = _sc_attn(dstv, srcv, v2, sco, pmax)

    d0 = denp[:_N * 4].reshape(_N, 4)
    d1 = denp[_D4:_D4 + _N * 4].reshape(_N, 4)
    h, st = _tc_head(nump[0], nump[1], d0, d1, sk, bskip.reshape(1, 256))
    out = _tc_bn(h, st, gamma.reshape(1, 256), beta.reshape(1, 256))
    return out
